# baseline (speedup 1.0000x reference)
import jax
import jax.numpy as jnp
from jax import lax
from jax.experimental import pallas as pl
from jax.experimental.pallas import tpu as pltpu
from jax.experimental.pallas import tpu_sc as plsc

DIM = 128
BATCH = 16384
NC = 2
NS = 16
L = 16
NW = NC * NS
BPW = BATCH // NW
CH = 128
NCH = BPW // CH
NBUF = 3
CHUNKS = ((0, CH), (CH, CH), (2 * CH, CH), (3 * CH, CH // 2), (3 * CH + CH // 2, CH // 2))


def _w2v_body(center_hbm, context_hbm, inw_hbm, outw_hbm, o_hbm,
              cidx_v, xidx_v, v_v, u_v, o_v, sem_v, sem_u, sem_i, sem_o):
    wid = lax.axis_index("s") * NC + lax.axis_index("c")

    def idx_slice(idx_ref, ci):
        start, nr = CHUNKS[ci]
        row, off = start // CH, start % CH
        if nr == CH:
            return idx_ref.at[row]
        return idx_ref.at[row, pl.ds(off, nr)]

    def gathers(ci):
        nr = CHUNKS[ci][1]
        buf = ci % NBUF
        cp_v = pltpu.async_copy(
            inw_hbm.at[idx_slice(cidx_v, ci)], v_v.at[buf, pl.ds(0, nr)], sem_v)
        cp_u = pltpu.async_copy(
            outw_hbm.at[idx_slice(xidx_v, ci)], u_v.at[buf, pl.ds(0, nr)], sem_u)
        return cp_v, cp_u

    i0c = pltpu.async_copy(center_hbm.at[wid, 0], cidx_v.at[0], sem_i)
    i0x = pltpu.async_copy(context_hbm.at[wid, 0], xidx_v.at[0], sem_i)
    irc = pltpu.async_copy(
        center_hbm.at[wid, pl.ds(1, NCH - 1)], cidx_v.at[pl.ds(1, NCH - 1)], sem_o)
    irx = pltpu.async_copy(
        context_hbm.at[wid, pl.ds(1, NCH - 1)], xidx_v.at[pl.ds(1, NCH - 1)], sem_o)
    i0c.wait()
    i0x.wait()

    cp = [None] * len(CHUNKS)
    cp[0] = gathers(0)
    irc.wait()
    irx.wait()
    cp[1] = gathers(1)
    cp[2] = gathers(2)

    out_cp = [None] * len(CHUNKS)
    lane = lax.iota(jnp.int32, L)
    RSUB = 4

    def _perm(x, k):
        return x.at[lane ^ k].get(mode="promise_in_bounds")

    def _merge(a, b, k):
        bit = (lane & k) == 0
        return jnp.where(bit, a, b) + jnp.where(bit, _perm(a, k), _perm(b, k))

    for ci, (start, nr) in enumerate(CHUNKS):
        buf = ci % NBUF
        cp[ci][0].wait()
        cp[ci][1].wait()

        def group_body(g, _, buf=buf, start=start):
            def sub_body(s, res, buf=buf):
                accs = []
                for jj in range(RSUB):
                    r = g * L + s * RSUB + jj
                    acc = v_v[buf, r, pl.ds(0, L)] * u_v[buf, r, pl.ds(0, L)]
                    for k in range(1, DIM // L):
                        acc = acc + v_v[buf, r, pl.ds(k * L, L)] * u_v[buf, r, pl.ds(k * L, L)]
                    accs.append(acc)
                t = _merge(_merge(accs[0], accs[1], 1), _merge(accs[2], accs[3], 1), 2)
                t = t + _perm(t, 4)
                t = t + _perm(t, 8)
                return jnp.where((lane >> 2) == s, t, res)

            res = lax.fori_loop(0, L // RSUB, sub_body, jnp.zeros((L,), jnp.float32))
            o_v[pl.ds(start + g * L, L)] = res
            return 0

        lax.fori_loop(0, nr // L, group_body, 0)
        out_cp[ci] = pltpu.async_copy(
            o_v.at[pl.ds(start, nr)], o_hbm.at[pl.ds(wid * BPW + start, nr)], sem_o)
        nxt = ci + NBUF
        if nxt < len(CHUNKS):
            cp[nxt] = gathers(nxt)

    for ci in range(len(CHUNKS)):
        out_cp[ci].wait()


def kernel(center_ids, context_ids, in_embed_w, out_embed_w):
    center_r = center_ids.reshape(NW, NCH, CH).astype(jnp.int32)
    context_r = context_ids.reshape(NW, NCH, CH).astype(jnp.int32)

    mesh = plsc.VectorSubcoreMesh(core_axis_name="c", subcore_axis_name="s")
    run = pl.kernel(
        _w2v_body,
        mesh=mesh,
        out_type=jax.ShapeDtypeStruct((BATCH,), jnp.float32),
        scratch_types=[
            pltpu.VMEM((NCH, CH), jnp.int32),
            pltpu.VMEM((NCH, CH), jnp.int32),
            pltpu.VMEM((NBUF, CH, DIM), jnp.float32),
            pltpu.VMEM((NBUF, CH, DIM), jnp.float32),
            pltpu.VMEM((BPW,), jnp.float32),
            pltpu.SemaphoreType.DMA,
            pltpu.SemaphoreType.DMA,
            pltpu.SemaphoreType.DMA,
            pltpu.SemaphoreType.DMA,
        ],
    )
    return run(center_r, context_r, in_embed_w, out_embed_w)


# speedup vs baseline: 1.0230x; 1.0230x over previous
import jax
import jax.numpy as jnp
from jax import lax
from jax.experimental import pallas as pl
from jax.experimental.pallas import tpu as pltpu
from jax.experimental.pallas import tpu_sc as plsc

DIM = 128
BATCH = 16384
NC = 2
NS = 16
L = 16
NW = NC * NS
BPW = BATCH // NW
CH = 128
NCH = BPW // CH
NBUF = 3


def _w2v_body(center_hbm, context_hbm, inw_hbm, outw_hbm, o_hbm,
              cidx_v, xidx_v, v_v, u_v, o_v, sem_v, sem_u, sem_i, sem_o):
    wid = lax.axis_index("s") * NC + lax.axis_index("c")

    def gathers(ci):
        buf = ci % NBUF
        cp_v = pltpu.async_copy(inw_hbm.at[cidx_v.at[ci]], v_v.at[buf], sem_v)
        cp_u = pltpu.async_copy(outw_hbm.at[xidx_v.at[ci]], u_v.at[buf], sem_u)
        return cp_v, cp_u

    i0c = pltpu.async_copy(center_hbm.at[wid, 0], cidx_v.at[0], sem_i)
    i0x = pltpu.async_copy(context_hbm.at[wid, 0], xidx_v.at[0], sem_i)
    irc = pltpu.async_copy(
        center_hbm.at[wid, pl.ds(1, NCH - 1)], cidx_v.at[pl.ds(1, NCH - 1)], sem_o)
    irx = pltpu.async_copy(
        context_hbm.at[wid, pl.ds(1, NCH - 1)], xidx_v.at[pl.ds(1, NCH - 1)], sem_o)
    i0c.wait()
    i0x.wait()

    cp = [None] * NCH
    cp[0] = gathers(0)
    irc.wait()
    irx.wait()
    for c in range(1, NBUF - 1):
        cp[c] = gathers(c)

    out_cp = [None] * NCH
    lane = lax.iota(jnp.int32, L)
    RSUB = 4

    def _perm(x, k):
        return x.at[lane ^ k].get(mode="promise_in_bounds")

    def _merge(a, b, k):
        bit = (lane & k) == 0
        return jnp.where(bit, a, b) + jnp.where(bit, _perm(a, k), _perm(b, k))

    for ci in range(NCH):
        buf = ci % NBUF
        n = ci + NBUF - 1
        if n < NCH:
            cp[n] = gathers(n)
        cp[ci][0].wait()
        cp[ci][1].wait()

        def group_body(g, _, buf=buf, ci=ci):
            def sub_body(s, res, buf=buf):
                accs = []
                for jj in range(RSUB):
                    r = g * L + s * RSUB + jj
                    acc = v_v[buf, r, pl.ds(0, L)] * u_v[buf, r, pl.ds(0, L)]
                    for k in range(1, DIM // L):
                        acc = acc + v_v[buf, r, pl.ds(k * L, L)] * u_v[buf, r, pl.ds(k * L, L)]
                    accs.append(acc)
                t = _merge(_merge(accs[0], accs[1], 1), _merge(accs[2], accs[3], 1), 2)
                t = t + _perm(t, 4)
                t = t + _perm(t, 8)
                return jnp.where((lane >> 2) == s, t, res)

            res = lax.fori_loop(0, L // RSUB, sub_body, jnp.zeros((L,), jnp.float32))
            o_v[pl.ds(ci * CH + g * L, L)] = res
            return 0

        lax.fori_loop(0, CH // L, group_body, 0)
        out_cp[ci] = pltpu.async_copy(
            o_v.at[pl.ds(ci * CH, CH)], o_hbm.at[pl.ds(wid * BPW + ci * CH, CH)], sem_o)

    for ci in range(NCH):
        out_cp[ci].wait()


def kernel(center_ids, context_ids, in_embed_w, out_embed_w):
    center_r = center_ids.reshape(NW, NCH, CH).astype(jnp.int32)
    context_r = context_ids.reshape(NW, NCH, CH).astype(jnp.int32)

    mesh = plsc.VectorSubcoreMesh(core_axis_name="c", subcore_axis_name="s")
    run = pl.kernel(
        _w2v_body,
        mesh=mesh,
        out_type=jax.ShapeDtypeStruct((BATCH,), jnp.float32),
        scratch_types=[
            pltpu.VMEM((NCH, CH), jnp.int32),
            pltpu.VMEM((NCH, CH), jnp.int32),
            pltpu.VMEM((NBUF, CH, DIM), jnp.float32),
            pltpu.VMEM((NBUF, CH, DIM), jnp.float32),
            pltpu.VMEM((BPW,), jnp.float32),
            pltpu.SemaphoreType.DMA,
            pltpu.SemaphoreType.DMA,
            pltpu.SemaphoreType.DMA,
            pltpu.SemaphoreType.DMA,
        ],
    )
    return run(center_r, context_r, in_embed_w, out_embed_w)
